# Initial kernel scaffold; baseline (speedup 1.0000x reference)
import jax
import jax.numpy as jnp
from jax.experimental import pallas as pl


def kernel(fmap, patches_f, coords, coords_phi, poses, i_idx, j_idx):
    raise NotImplementedError("write your pallas kernel here")


# speedup vs baseline: 17.7738x; 17.7738x over previous
import functools

import numpy as np
import jax
import jax.numpy as jnp
from jax.experimental import pallas as pl
from jax.experimental.pallas import tpu as pltpu

B = 2; N = 4; P = 256; C = 64; H = 256; W = 256
DS = 4; PS = 5; CN = 3; SS = CN + PS - 1; TW = 2
R_MIN = 0.5; R_MAX = 30.0
FOV_H = 90.0 * np.pi / 180.0
FOV_V = 20.0 * np.pi / 180.0
EPS = 1e-2

BN = B * N
E = 2 * B * P * (N * TW - TW * (TW + 1) // 2)
NB = E // P
PAD1 = 8
PAD2 = 16
H1 = H + 2 * PAD1; W1 = W + 2 * PAD1
H2 = H // DS + 2 * PAD2; W2 = W // DS + 2 * PAD2
GRID_LO = -(SS // 2)


def _block_schedule():
    js = []
    for _ in range(B):
        per = []
        for t in range(N):
            for k in range(1, TW + 1):
                if t - k >= 0:
                    per.append(t - k)
                    per.append(t)
        js.append(per)
    js = np.asarray(js)
    js = js + (np.arange(B) * N)[:, None]
    flat = js.reshape(-1)
    return np.argsort(flat, kind="stable").astype(np.int32)


_SCHED = _block_schedule()


def _pool_body(x_ref, o_ref):
    x = x_ref[0]
    x = x.reshape(64 // DS, DS, W // DS, DS, C)
    o_ref[0] = jnp.mean(x, axis=(1, 3))


def _pool(fmap_t):
    return pl.pallas_call(
        _pool_body,
        grid=(BN, H // 64),
        in_specs=[pl.BlockSpec((1, 64, W, C), lambda b, h: (b, h, 0, 0))],
        out_specs=pl.BlockSpec((1, 64 // DS, W // DS, C),
                               lambda b, h: (b, h, 0, 0)),
        out_shape=jax.ShapeDtypeStruct((BN, H // DS, W // DS, C),
                                       jnp.float32),
    )(fmap_t)


def _qrot_terms(qx, qy, qz, qw, vx, vy, vz):
    tx = 2.0 * (qy * vz - qz * vy)
    ty = 2.0 * (qz * vx - qx * vz)
    tz = 2.0 * (qx * vy - qy * vx)
    ox = vx + qw * tx + (qy * tz - qz * ty)
    oy = vy + qw * ty + (qz * tx - qx * tz)
    oz = vz + qw * tz + (qx * ty - qy * tx)
    return ox, oy, oz


NSEM = 16


def _edge_body(blk_ref, basei_ref, st_ref, pi_ref, pj_ref, basef_ref,
               patch_ref, f1_ref, f2_ref, out_ref, s1_ref, s2_ref, sem_ref):
    g = pl.program_id(0)

    st = st_ref[0]
    cx = st[:, 0:1]; cy = st[:, 1:2]; ph0 = st[:, 2:3]
    r = cy / H * (R_MAX - R_MIN) + R_MIN
    th = (cx / W - 0.5) * FOV_H * (np.pi / 180.0)

    cph = jnp.cos(ph0)
    vx = r * cph * jnp.cos(th)
    vy = r * cph * jnp.sin(th)
    vz = r * jnp.sin(ph0)

    ps = pi_ref[0]
    pt = pj_ref[0]
    qsx = ps[:, 3:4]; qsy = ps[:, 4:5]; qsz = ps[:, 5:6]; qsw = ps[:, 6:7]
    ns = jax.lax.rsqrt(qsx * qsx + qsy * qsy + qsz * qsz + qsw * qsw)
    qsx = qsx * ns; qsy = qsy * ns; qsz = qsz * ns; qsw = qsw * ns
    qtx = pt[:, 3:4]; qty = pt[:, 4:5]; qtz = pt[:, 5:6]; qtw = pt[:, 6:7]
    nt = jax.lax.rsqrt(qtx * qtx + qty * qty + qtz * qtz + qtw * qtw)
    qtx = qtx * nt; qty = qty * nt; qtz = qtz * nt; qtw = qtw * nt

    wx_, wy_, wz_ = _qrot_terms(qsx, qsy, qsz, qsw, vx, vy, vz)
    wx_ = wx_ + ps[:, 0:1]; wy_ = wy_ + ps[:, 1:2]; wz_ = wz_ + ps[:, 2:3]
    dx_ = wx_ - pt[:, 0:1]; dy_ = wy_ - pt[:, 1:2]; dz_ = wz_ - pt[:, 2:3]
    lx, ly, lz = _qrot_terms(-qtx, -qty, -qtz, qtw, dx_, dy_, dz_)

    r2 = jnp.sqrt(lx * lx + ly * ly + lz * lz)
    th2 = jnp.arctan2(ly, lx)
    sph = jnp.clip(lz / jnp.maximum(r2, 1e-8), -1.0, 1.0)

    oor = ((r2 < R_MIN - EPS) | (r2 > R_MAX + EPS)
           | (jnp.abs(th2) > FOV_H / 2 + EPS)
           | (jnp.abs(sph) > np.sin(FOV_V / 2 + EPS)))
    mask = jnp.where(oor, 0.0, 1.0)

    xc = (th2 * (180.0 / np.pi) / FOV_H + 0.5) * W
    yc = (r2 - R_MIN) / (R_MAX - R_MIN) * H

    bf = basef_ref[0]
    w1x = (xc + GRID_LO) - bf[:, 0:1]
    w1y = (yc + GRID_LO) - bf[:, 1:2]
    w2x = (xc / DS + GRID_LO) - bf[:, 2:3]
    w2y = (yc / DS + GRID_LO) - bf[:, 3:4]

    jv = blk_ref[g, 1]

    def mk1(e):
        bx1 = basei_ref[0, g, e]
        by1 = basei_ref[1, g, e]
        return pltpu.make_async_copy(
            f1_ref.at[jv, pl.ds(by1, SS + 1), pl.ds(bx1, SS + 1), :],
            s1_ref.at[e],
            sem_ref.at[0, jax.lax.rem(e, NSEM)])

    def mk2(e):
        bx2 = basei_ref[2, g, e]
        by2 = basei_ref[3, g, e]
        return pltpu.make_async_copy(
            f2_ref.at[jv, pl.ds(by2, SS + 1), pl.ds(bx2, SS + 1), :],
            s2_ref.at[e],
            sem_ref.at[1, jax.lax.rem(e, NSEM)])

    def load_one(e, carry):
        @pl.when(e >= NSEM)
        def _wait_old():
            mk1(e - NSEM).wait()
            mk2(e - NSEM).wait()
        mk1(e).start()
        mk2(e).start()
        return carry

    jax.lax.fori_loop(0, P, load_one, 0)

    def drain_one(e, carry):
        mk1(e).wait()
        mk2(e).wait()
        return carry

    jax.lax.fori_loop(P - NSEM, P, drain_one, 0)

    def bilin(s_ref, wxv, wyv):
        s = s_ref[:]
        a00 = ((1.0 - wxv) * (1.0 - wyv)).reshape(P, 1, 1, 1)
        a01 = (wxv * (1.0 - wyv)).reshape(P, 1, 1, 1)
        a10 = ((1.0 - wxv) * wyv).reshape(P, 1, 1, 1)
        a11 = (wxv * wyv).reshape(P, 1, 1, 1)
        return (s[:, :SS, :SS, :] * a00 + s[:, :SS, 1:, :] * a01
                + s[:, 1:, :SS, :] * a10 + s[:, 1:, 1:, :] * a11)

    samp1 = bilin(s1_ref, w1x, w1y)
    samp2 = bilin(s2_ref, w2x, w2y)

    src = patch_ref[0]
    cols = []
    for samp in (samp1, samp2):
        for wy in range(CN):
            for wx in range(CN):
                prod = samp[:, wy:wy + PS, wx:wx + PS, :] * src
                cols.append(jnp.sum(prod, axis=(1, 2, 3)).reshape(P, 1))
    out_ref[0] = jnp.concatenate(cols, axis=1) * mask


def _edge_call(blk, basei, st, poses_r, basef, patches_t, f1p, f2p):
    spec = pltpu.PrefetchScalarGridSpec(
        num_scalar_prefetch=2,
        grid=(NB,),
        in_specs=[
            pl.BlockSpec((1, P, 3), lambda g, b, bi: (b[g, 0], 0, 0)),
            pl.BlockSpec((1, 1, 7), lambda g, b, bi: (b[g, 0], 0, 0)),
            pl.BlockSpec((1, 1, 7), lambda g, b, bi: (b[g, 1], 0, 0)),
            pl.BlockSpec((1, P, 4), lambda g, b, bi: (g, 0, 0)),
            pl.BlockSpec((1, P, PS, PS, C),
                         lambda g, b, bi: (b[g, 0], 0, 0, 0, 0)),
            pl.BlockSpec(memory_space=pl.ANY),
            pl.BlockSpec(memory_space=pl.ANY),
        ],
        out_specs=pl.BlockSpec((1, P, 2 * CN * CN),
                               lambda g, b, bi: (b[g, 2], 0, 0)),
        scratch_shapes=[
            pltpu.VMEM((P, SS + 1, SS + 1, C), jnp.float32),
            pltpu.VMEM((P, SS + 1, SS + 1, C), jnp.float32),
            pltpu.SemaphoreType.DMA((2, NSEM)),
        ],
    )
    return pl.pallas_call(
        _edge_body,
        grid_spec=spec,
        out_shape=jax.ShapeDtypeStruct((NB, P, 2 * CN * CN), jnp.float32),
        compiler_params=pltpu.CompilerParams(
            vmem_limit_bytes=128 * 1024 * 1024),
    )(blk, basei, st, poses_r, poses_r, basef, patches_t, f1p, f2p)


def _host_centers(coords, coords_phi, poses, i_idx, j_idx):
    pts = B * N * P
    c2 = coords.reshape(pts, 2)
    r = c2[:, 1] / H * (R_MAX - R_MIN) + R_MIN
    th = (c2[:, 0] / W - 0.5) * FOV_H * (np.pi / 180.0)
    ph0 = coords_phi.reshape(pts)
    r = r[i_idx]; th = th[i_idx]; ph0 = ph0[i_idx]

    pf = poses.reshape(BN, 7)
    ps = pf[i_idx // P]
    pt = pf[j_idx]
    cph = jnp.cos(ph0)
    vx = r * cph * jnp.cos(th); vy = r * cph * jnp.sin(th)
    vz = r * jnp.sin(ph0)
    qs = ps[:, 3:7]; qs = qs / jnp.linalg.norm(qs, axis=1, keepdims=True)
    qt = pt[:, 3:7]; qt = qt / jnp.linalg.norm(qt, axis=1, keepdims=True)
    wx, wy, wz = _qrot_terms(qs[:, 0], qs[:, 1], qs[:, 2], qs[:, 3],
                             vx, vy, vz)
    wx = wx + ps[:, 0]; wy = wy + ps[:, 1]; wz = wz + ps[:, 2]
    dx = wx - pt[:, 0]; dy = wy - pt[:, 1]; dz = wz - pt[:, 2]
    lx, ly, lz = _qrot_terms(-qt[:, 0], -qt[:, 1], -qt[:, 2], qt[:, 3],
                             dx, dy, dz)
    r2 = jnp.sqrt(lx * lx + ly * ly + lz * lz)
    th2 = jnp.arctan2(ly, lx)
    xc = (th2 * (180.0 / np.pi) / FOV_H + 0.5) * W
    yc = (r2 - R_MIN) / (R_MAX - R_MIN) * H
    return xc, yc


@jax.jit
def kernel(fmap, patches_f, coords, coords_phi, poses, i_idx, j_idx):
    i_idx = i_idx.astype(jnp.int32)
    j_idx = j_idx.astype(jnp.int32)
    sched = jnp.asarray(_SCHED)

    fmap_t = jnp.transpose(fmap.reshape(BN, C, H, W), (0, 2, 3, 1))
    f2t = _pool(fmap_t)
    f1p = jnp.pad(fmap_t, ((0, 0), (PAD1, PAD1), (PAD1, PAD1), (0, 0)))
    f2p = jnp.pad(f2t, ((0, 0), (PAD2, PAD2), (PAD2, PAD2), (0, 0)))
    patches_t = jnp.transpose(
        patches_f.reshape(BN, P, C, PS, PS), (0, 1, 3, 4, 2))
    st = jnp.concatenate([
        coords.reshape(BN, P, 2), coords_phi.reshape(BN, P, 1)], axis=2)
    poses_r = poses.reshape(BN, 1, 7)

    ib = i_idx.reshape(NB, P)[:, 0] // P
    jb = j_idx.reshape(NB, P)[:, 0]
    blk = jnp.stack([ib[sched], jb[sched], sched], axis=1).astype(jnp.int32)

    xc, yc = _host_centers(coords, coords_phi, poses, i_idx, j_idx)
    fx1 = jnp.floor(xc + GRID_LO); fy1 = jnp.floor(yc + GRID_LO)
    fx2 = jnp.floor(xc / DS + GRID_LO); fy2 = jnp.floor(yc / DS + GRID_LO)
    bx1 = jnp.clip(fx1.astype(jnp.int32) + PAD1, 0, W1 - (SS + 1))
    by1 = jnp.clip(fy1.astype(jnp.int32) + PAD1, 0, H1 - (SS + 1))
    bx2 = jnp.clip(fx2.astype(jnp.int32) + PAD2, 0, W2 - (SS + 1))
    by2 = jnp.clip(fy2.astype(jnp.int32) + PAD2, 0, H2 - (SS + 1))
    basei = jnp.stack([bx1, by1, bx2, by2], axis=0).reshape(4, NB, P)
    basei = basei[:, sched]
    basef = jnp.stack([fx1, fy1, fx2, fy2], axis=1).reshape(NB, P, 4)
    basef = basef[sched]

    out = _edge_call(blk, basei, st, poses_r, basef, patches_t, f1p, f2p)
    return out.reshape(E, 2 * CN * CN)
